# baseline (speedup 1.0000x reference)
import functools

import jax
import jax.numpy as jnp
from jax import lax
from jax.experimental import pallas as pl
from jax.experimental.pallas import tpu as pltpu
from jax.experimental.pallas import tpu_sc as plsc

B, L, D = 4, 4096, 1024
POS_N = L * D
TOT = B * POS_N
NC, NS = 2, 16
NW = NC * NS
PER_W = TOT // NW
CHUNK = 32768
NCHUNK = PER_W // CHUNK


def _sc_add(x_hbm, pos_hbm, out_hbm, xbuf, pbuf):
    wid = lax.axis_index("s") * NC + lax.axis_index("c")
    base = wid * PER_W
    pbase = lax.rem(base, POS_N)

    def chunk_body(c, _):
        off = base + c * CHUNK
        poff = pbase + c * CHUNK
        pltpu.sync_copy(x_hbm.at[pl.ds(off, CHUNK)], xbuf)
        pltpu.sync_copy(pos_hbm.at[pl.ds(poff, CHUNK)], pbuf)

        def add_body(j, _):
            s = pl.ds(j * 16, 16)
            xbuf[s] = xbuf[s] + pbuf[s]
            return 0

        lax.fori_loop(0, CHUNK // 16, add_body, 0)
        pltpu.sync_copy(xbuf, out_hbm.at[pl.ds(off, CHUNK)])
        return 0

    lax.fori_loop(0, NCHUNK, chunk_body, 0)


_sc_kernel = functools.partial(
    pl.kernel,
    mesh=plsc.VectorSubcoreMesh(core_axis_name="c", subcore_axis_name="s"),
    out_type=jax.ShapeDtypeStruct((TOT,), jnp.float32),
    scratch_types=[
        pltpu.VMEM((CHUNK,), jnp.float32),
        pltpu.VMEM((CHUNK,), jnp.float32),
    ],
)(_sc_add)


def kernel(x, pos_table):
    out = _sc_kernel(x.reshape(-1), pos_table.reshape(-1))
    return out.reshape(x.shape)


# speedup vs baseline: 1.5346x; 1.5346x over previous
import functools

import jax
import jax.numpy as jnp
from jax import lax
from jax.experimental import pallas as pl
from jax.experimental.pallas import tpu as pltpu
from jax.experimental.pallas import tpu_sc as plsc

B, L, D = 4, 4096, 1024
POS_N = L * D
TOT = B * POS_N
NC, NS = 2, 16
NW = NC * NS
PER_W = TOT // NW
CHUNK = 16384
NCHUNK = PER_W // CHUNK


def _sc_add(x_hbm, pos_hbm, out_hbm, xbuf, pbuf, obuf, xs0, xs1, ps0, ps1,
            os0, os1):
    xsem = (xs0, xs1)
    psem = (ps0, ps1)
    osem = (os0, os1)
    wid = lax.axis_index("s") * NC + lax.axis_index("c")
    base = wid * PER_W
    pbase = lax.rem(base, POS_N)

    def start_in(cc, b):
        off = base + cc * CHUNK
        poff = pbase + cc * CHUNK
        pltpu.async_copy(x_hbm.at[pl.ds(off, CHUNK)], xbuf.at[b], xsem[b])
        pltpu.async_copy(pos_hbm.at[pl.ds(poff, CHUNK)], pbuf.at[b], psem[b])

    def wait_in(cc, b):
        off = base + cc * CHUNK
        poff = pbase + cc * CHUNK
        pltpu.make_async_copy(x_hbm.at[pl.ds(off, CHUNK)], xbuf.at[b],
                              xsem[b]).wait()
        pltpu.make_async_copy(pos_hbm.at[pl.ds(poff, CHUNK)], pbuf.at[b],
                              psem[b]).wait()

    def wait_out(cc, b):
        off = base + cc * CHUNK
        pltpu.make_async_copy(obuf.at[b], out_hbm.at[pl.ds(off, CHUNK)],
                              osem[b]).wait()

    start_in(0, 0)
    start_in(1, 1)

    @pl.loop(0, NCHUNK, step=2)
    def _(c):
        for b in range(2):
            cc = c + b
            wait_in(cc, b)

            @pl.when(cc >= 2)
            def _():
                wait_out(cc - 2, b)

            @plsc.parallel_loop(0, CHUNK // 16, unroll=8)
            def _(j):
                s = pl.ds(j * 16, 16)
                obuf[b, s] = xbuf[b, s] + pbuf[b, s]

            off = base + cc * CHUNK
            pltpu.async_copy(obuf.at[b], out_hbm.at[pl.ds(off, CHUNK)],
                             osem[b])

            @pl.when(cc + 2 < NCHUNK)
            def _():
                start_in(cc + 2, b)

    wait_out(NCHUNK - 2, 0)
    wait_out(NCHUNK - 1, 1)


_sc_kernel = functools.partial(
    pl.kernel,
    mesh=plsc.VectorSubcoreMesh(core_axis_name="c", subcore_axis_name="s"),
    out_type=jax.ShapeDtypeStruct((TOT,), jnp.float32),
    scratch_types=[
        pltpu.VMEM((2, CHUNK), jnp.float32),
        pltpu.VMEM((2, CHUNK), jnp.float32),
        pltpu.VMEM((2, CHUNK), jnp.float32),
        pltpu.SemaphoreType.DMA,
        pltpu.SemaphoreType.DMA,
        pltpu.SemaphoreType.DMA,
        pltpu.SemaphoreType.DMA,
        pltpu.SemaphoreType.DMA,
        pltpu.SemaphoreType.DMA,
    ],
)(_sc_add)


def kernel(x, pos_table):
    out = _sc_kernel(x.reshape(-1), pos_table.reshape(-1))
    return out.reshape(x.shape)


# speedup vs baseline: 4.2873x; 2.7937x over previous
import functools

import jax
import jax.numpy as jnp
from jax import lax
from jax.experimental import pallas as pl
from jax.experimental.pallas import tpu as pltpu
from jax.experimental.pallas import tpu_sc as plsc

B, L, D = 4, 4096, 1024
NC, NS = 2, 16
NW = NC * NS
ROWS_W = B * L // NW
W_PER_B = L // ROWS_W
R = 16
NCHUNK = ROWS_W // R


def _sc_add(x_hbm, pos_hbm, out_hbm, xbuf, pbuf, obuf, xs0, xs1, ps0, ps1,
            os0, os1):
    xsem = (xs0, xs1)
    psem = (ps0, ps1)
    osem = (os0, os1)
    wid = lax.axis_index("s") * NC + lax.axis_index("c")
    bi = wid // W_PER_B
    lbase = (wid % W_PER_B) * ROWS_W

    def start_in(cc, b):
        l0 = lbase + cc * R
        pltpu.async_copy(x_hbm.at[bi, pl.ds(l0, R), :], xbuf.at[b], xsem[b])
        pltpu.async_copy(pos_hbm.at[pl.ds(l0, R), :], pbuf.at[b], psem[b])

    def wait_in(cc, b):
        l0 = lbase + cc * R
        pltpu.make_async_copy(x_hbm.at[bi, pl.ds(l0, R), :], xbuf.at[b],
                              xsem[b]).wait()
        pltpu.make_async_copy(pos_hbm.at[pl.ds(l0, R), :], pbuf.at[b],
                              psem[b]).wait()

    def wait_out(cc, b):
        l0 = lbase + cc * R
        pltpu.make_async_copy(obuf.at[b], out_hbm.at[bi, pl.ds(l0, R), :],
                              osem[b]).wait()

    start_in(0, 0)
    start_in(1, 1)

    @pl.loop(0, NCHUNK, step=2)
    def _(c):
        for b in range(2):
            cc = c + b
            wait_in(cc, b)

            @pl.when(cc >= 2)
            def _():
                wait_out(cc - 2, b)

            @plsc.parallel_loop(0, R * D // 16, unroll=8)
            def _(j):
                r = j >> (10 - 4)
                col = (j & (D // 16 - 1)) * 16
                s = pl.ds(col, 16)
                obuf[b, r, s] = xbuf[b, r, s] + pbuf[b, r, s]

            l0 = lbase + cc * R
            pltpu.async_copy(obuf.at[b], out_hbm.at[bi, pl.ds(l0, R), :],
                             osem[b])

            @pl.when(cc + 2 < NCHUNK)
            def _():
                start_in(cc + 2, b)

    wait_out(NCHUNK - 2, 0)
    wait_out(NCHUNK - 1, 1)


_sc_kernel = functools.partial(
    pl.kernel,
    mesh=plsc.VectorSubcoreMesh(core_axis_name="c", subcore_axis_name="s"),
    out_type=jax.ShapeDtypeStruct((B, L, D), jnp.float32),
    scratch_types=[
        pltpu.VMEM((2, R, D), jnp.float32),
        pltpu.VMEM((2, R, D), jnp.float32),
        pltpu.VMEM((2, R, D), jnp.float32),
        pltpu.SemaphoreType.DMA,
        pltpu.SemaphoreType.DMA,
        pltpu.SemaphoreType.DMA,
        pltpu.SemaphoreType.DMA,
        pltpu.SemaphoreType.DMA,
        pltpu.SemaphoreType.DMA,
    ],
)(_sc_add)


def kernel(x, pos_table):
    return _sc_kernel(x, pos_table)


# speedup vs baseline: 8.1848x; 1.9091x over previous
import jax
import jax.numpy as jnp
from jax.experimental import pallas as pl

BLOCK_L = 512


def _add_kernel(x_ref, pos_ref, o_ref):
    o_ref[...] = x_ref[...] + pos_ref[...][None, :, :]


def kernel(x, pos_table):
    B, L, D = x.shape
    return pl.pallas_call(
        _add_kernel,
        grid=(L // BLOCK_L,),
        in_specs=[
            pl.BlockSpec((B, BLOCK_L, D), lambda l: (0, l, 0)),
            pl.BlockSpec((BLOCK_L, D), lambda l: (l, 0)),
        ],
        out_specs=pl.BlockSpec((B, BLOCK_L, D), lambda l: (0, l, 0)),
        out_shape=jax.ShapeDtypeStruct((B, L, D), x.dtype),
    )(x, pos_table)
